# baseline (speedup 1.0000x reference)
import functools

import jax
import jax.numpy as jnp
from jax import lax
from jax.experimental import pallas as pl
from jax.experimental.pallas import tpu as pltpu
from jax.experimental.pallas import tpu_sc as plsc

_B = 4096
_F = 26
_VOCAB = 100000
_E = 16
_H1, _H2 = 64, 32
_S = 5
_DIN = _F * _E

_CHUNK = 128
_KSC = 18
_ROWS = _B * _KSC
_NCHT = _ROWS // _CHUNK
_NSEM = 8


def _sc_gather_body(nch, utab, itab, uidx, iidx, uout, iout,
                    uidx_v, iidx_v, urows_v, irows_v, sems):
    info = plsc.get_sparse_core_info()
    nc = info.num_cores
    wid = lax.axis_index("s") * nc + lax.axis_index("c")

    pltpu.sync_copy(uidx.at[wid], uidx_v)
    pltpu.sync_copy(iidx.at[wid], iidx_v)

    def fire(j, c):
        pltpu.async_copy(utab.at[uidx_v.at[j]], urows_v.at[j],
                         sems.at[lax.rem(2 * j, _NSEM)])
        pltpu.async_copy(itab.at[iidx_v.at[j]], irows_v.at[j],
                         sems.at[lax.rem(2 * j + 1, _NSEM)])
        return c

    lax.fori_loop(0, nch, fire, 0)

    def drain(j, c):
        pltpu.make_async_copy(utab.at[uidx_v.at[j]], urows_v.at[j],
                              sems.at[lax.rem(2 * j, _NSEM)]).wait()
        pltpu.make_async_copy(itab.at[iidx_v.at[j]], irows_v.at[j],
                              sems.at[lax.rem(2 * j + 1, _NSEM)]).wait()
        return c

    lax.fori_loop(0, nch, drain, 0)

    pltpu.sync_copy(urows_v, uout.at[wid])
    pltpu.sync_copy(irows_v, iout.at[wid])


def _sc_gather(utab, itab, uidx2, iidx2):
    info = plsc.get_sparse_core_info()
    nw = info.num_cores * info.num_subcores
    nch = _NCHT // nw
    mesh = plsc.VectorSubcoreMesh(core_axis_name="c", subcore_axis_name="s")
    out_t = jax.ShapeDtypeStruct((nw, nch, _CHUNK, _E), jnp.float32)
    f = pl.kernel(
        functools.partial(_sc_gather_body, nch),
        out_type=(out_t, out_t),
        mesh=mesh,
        scratch_types=[
            pltpu.VMEM((nch, _CHUNK), jnp.int32),
            pltpu.VMEM((nch, _CHUNK), jnp.int32),
            pltpu.VMEM((nch, _CHUNK, _E), jnp.float32),
            pltpu.VMEM((nch, _CHUNK, _E), jnp.float32),
            pltpu.SemaphoreType.DMA((_NSEM,)),
        ],
        compiler_params=pltpu.CompilerParams(use_tc_tiling_on_sc=False),
    )
    return f(utab, itab, uidx2, iidx2)


_TB = 256


def _towers_body(ux, ix, wu1, bu1, wu2, bu2, wi1, bi1, wi2, bi2, uo, io):
    u = jnp.maximum(jnp.dot(ux[...].astype(jnp.float32), wu1[...],
                            preferred_element_type=jnp.float32) + bu1[...], 0.0)
    u = jnp.maximum(jnp.dot(u, wu2[...],
                            preferred_element_type=jnp.float32) + bu2[...], 0.0)
    uo[...] = u
    it = jnp.maximum(jnp.dot(ix[...].astype(jnp.float32), wi1[...],
                             preferred_element_type=jnp.float32) + bi1[...], 0.0)
    it = jnp.maximum(jnp.dot(it, wi2[...],
                             preferred_element_type=jnp.float32) + bi2[...], 0.0)
    io[...] = it


def _towers(ux, ix, wu1, bu1, wu2, bu2, wi1, bi1, wi2, bi2):
    grid = (_B // _TB,)
    row_spec = pl.BlockSpec((_TB, _DIN), lambda t: (t, 0))
    out_spec = pl.BlockSpec((_TB, _H2), lambda t: (t, 0))

    def full(shape):
        return pl.BlockSpec(shape, lambda t: tuple(0 for _ in shape))

    return pl.pallas_call(
        _towers_body,
        grid=grid,
        in_specs=[row_spec, row_spec,
                  full((_DIN, _H1)), full((1, _H1)), full((_H1, _H2)), full((1, _H2)),
                  full((_DIN, _H1)), full((1, _H1)), full((_H1, _H2)), full((1, _H2))],
        out_specs=[out_spec, out_spec],
        out_shape=[jax.ShapeDtypeStruct((_B, _H2), jnp.float32),
                   jax.ShapeDtypeStruct((_B, _H2), jnp.float32)],
    )(ux, ix, wu1, bu1, wu2, bu2, wi1, bi1, wi2, bi2)


def _loss_body(uo, io, lbl, smp, loss):
    u = uo[...]
    it = io[...]
    lblc = lbl[...]
    cols = lax.broadcasted_iota(jnp.int32, (_TB, _B), 1)
    onehot = (cols == lblc).astype(jnp.float32)
    true_w = jnp.dot(onehot, it, preferred_element_type=jnp.float32)
    t = jnp.sum(u * true_w, axis=1, keepdims=True)

    svals = smp[...][:, 0:1]
    scols = lax.broadcasted_iota(jnp.int32, (8, _B), 1)
    smat = (scols == svals).astype(jnp.float32)
    sw = jnp.dot(smat, it, preferred_element_type=jnp.float32)

    m = t
    sls = []
    for j in range(_S):
        slj = jnp.sum(u * sw[j:j + 1, :], axis=1, keepdims=True)
        sls.append(slj)
        m = jnp.maximum(m, slj)
    denom = jnp.exp(t - m)
    for slj in sls:
        denom = denom + jnp.exp(slj - m)
    loss[...] = jnp.log(denom) + m - t


def _loss(uo, io, lbl2, smp):
    grid = (_B // _TB,)
    return pl.pallas_call(
        _loss_body,
        grid=grid,
        in_specs=[pl.BlockSpec((_TB, _H2), lambda t: (t, 0)),
                  pl.BlockSpec((_B, _H2), lambda t: (0, 0)),
                  pl.BlockSpec((_TB, 1), lambda t: (t, 0)),
                  pl.BlockSpec((8, 128), lambda t: (0, 0))],
        out_specs=pl.BlockSpec((_TB, 1), lambda t: (t, 0)),
        out_shape=jax.ShapeDtypeStruct((_B, 1), jnp.float32),
    )(uo, io, lbl2, smp)


def kernel(user_sparse_inputs, user_dense_inputs, item_sparse_inputs,
           item_dense_inputs, labels, user_tables, item_tables,
           W_u1, b_u1, W_u2, b_u2, W_i1, b_i1, W_i2, b_i2):
    info = plsc.get_sparse_core_info()
    nw = info.num_cores * info.num_subcores
    nch = _NCHT // nw
    offs = (jnp.arange(_KSC, dtype=jnp.int32) * _VOCAB)[None, :]
    usp = user_sparse_inputs.astype(jnp.int32)
    isp = item_sparse_inputs.astype(jnp.int32)
    uidx = (usp[:, :_KSC] + offs).reshape(nw, nch, _CHUNK)
    iidx = (isp[:, :_KSC] + offs).reshape(nw, nch, _CHUNK)
    utab = user_tables[:_KSC].reshape(_KSC * _VOCAB, _E)
    itab = item_tables[:_KSC].reshape(_KSC * _VOCAB, _E)

    uemb, iemb = _sc_gather(utab, itab, uidx, iidx)

    def _tc_take(tabs, idx):
        embs = jax.vmap(lambda t, i: jnp.take(t, i, axis=0),
                        in_axes=(0, 1))(tabs, idx)
        return jnp.transpose(embs, (1, 0, 2)).reshape(_B, (_F - _KSC) * _E)

    ux = jnp.concatenate(
        [uemb.reshape(_B, _KSC * _E), _tc_take(user_tables[_KSC:], usp[:, _KSC:])],
        axis=1)
    ix = jnp.concatenate(
        [iemb.reshape(_B, _KSC * _E), _tc_take(item_tables[_KSC:], isp[:, _KSC:])],
        axis=1)

    uo, io = _towers(ux, ix, W_u1, b_u1.reshape(1, _H1), W_u2, b_u2.reshape(1, _H2),
                     W_i1, b_i1.reshape(1, _H1), W_i2, b_i2.reshape(1, _H2))

    sampled = jax.random.randint(jax.random.key(42), (_S,), 0, _B)
    smp = jnp.zeros((8, 128), jnp.int32).at[:_S, 0].set(sampled)
    lbl2 = labels.reshape(_B, 1).astype(jnp.int32)

    loss = _loss(uo, io, lbl2, smp)
    return loss.reshape(_B)


# speedup vs baseline: 1.1022x; 1.1022x over previous
import functools

import jax
import jax.numpy as jnp
from jax import lax
from jax.experimental import pallas as pl
from jax.experimental.pallas import tpu as pltpu
from jax.experimental.pallas import tpu_sc as plsc

_B = 4096
_F = 26
_VOCAB = 100000
_E = 16
_H1, _H2 = 64, 32
_S = 5
_DIN = _F * _E

_CHUNK = 128
_KSC = 16
_ROWS = _B * _KSC
_NCHT = _ROWS // _CHUNK
_NSEM = 8


def _sc_gather_body(nch, utab, itab, uidx, iidx, uout, iout,
                    uidx_v, iidx_v, urows_v, irows_v, sems):
    info = plsc.get_sparse_core_info()
    nc = info.num_cores
    wid = lax.axis_index("s") * nc + lax.axis_index("c")

    pltpu.sync_copy(uidx.at[wid], uidx_v)
    pltpu.sync_copy(iidx.at[wid], iidx_v)

    def fire(j, c):
        pltpu.async_copy(utab.at[uidx_v.at[j]], urows_v.at[j],
                         sems.at[lax.rem(2 * j, _NSEM)])
        pltpu.async_copy(itab.at[iidx_v.at[j]], irows_v.at[j],
                         sems.at[lax.rem(2 * j + 1, _NSEM)])
        return c

    lax.fori_loop(0, nch, fire, 0)

    def drain(j, c):
        pltpu.make_async_copy(utab.at[uidx_v.at[j]], urows_v.at[j],
                              sems.at[lax.rem(2 * j, _NSEM)]).wait()
        pltpu.make_async_copy(itab.at[iidx_v.at[j]], irows_v.at[j],
                              sems.at[lax.rem(2 * j + 1, _NSEM)]).wait()
        return c

    lax.fori_loop(0, nch, drain, 0)

    pltpu.sync_copy(urows_v, uout.at[wid])
    pltpu.sync_copy(irows_v, iout.at[wid])


def _sc_gather(utab, itab, uidx2, iidx2):
    info = plsc.get_sparse_core_info()
    nw = info.num_cores * info.num_subcores
    nch = _NCHT // nw
    mesh = plsc.VectorSubcoreMesh(core_axis_name="c", subcore_axis_name="s")
    out_t = jax.ShapeDtypeStruct((nw, nch, _CHUNK, _E), jnp.float32)
    f = pl.kernel(
        functools.partial(_sc_gather_body, nch),
        out_type=(out_t, out_t),
        mesh=mesh,
        scratch_types=[
            pltpu.VMEM((nch, _CHUNK), jnp.int32),
            pltpu.VMEM((nch, _CHUNK), jnp.int32),
            pltpu.VMEM((nch, _CHUNK, _E), jnp.float32),
            pltpu.VMEM((nch, _CHUNK, _E), jnp.float32),
            pltpu.SemaphoreType.DMA((_NSEM,)),
        ],
        compiler_params=pltpu.CompilerParams(use_tc_tiling_on_sc=False),
    )
    return f(utab, itab, uidx2, iidx2)


_TB = 256


def _towers_body(ux, ix, wu1, bu1, wu2, bu2, wi1, bi1, wi2, bi2, uo, io):
    u = jnp.maximum(jnp.dot(ux[...].astype(jnp.float32), wu1[...],
                            preferred_element_type=jnp.float32) + bu1[...], 0.0)
    u = jnp.maximum(jnp.dot(u, wu2[...],
                            preferred_element_type=jnp.float32) + bu2[...], 0.0)
    uo[...] = u
    it = jnp.maximum(jnp.dot(ix[...].astype(jnp.float32), wi1[...],
                             preferred_element_type=jnp.float32) + bi1[...], 0.0)
    it = jnp.maximum(jnp.dot(it, wi2[...],
                             preferred_element_type=jnp.float32) + bi2[...], 0.0)
    io[...] = it


def _towers(ux, ix, wu1, bu1, wu2, bu2, wi1, bi1, wi2, bi2):
    grid = (_B // _TB,)
    row_spec = pl.BlockSpec((_TB, _DIN), lambda t: (t, 0))
    out_spec = pl.BlockSpec((_TB, _H2), lambda t: (t, 0))

    def full(shape):
        return pl.BlockSpec(shape, lambda t: tuple(0 for _ in shape))

    return pl.pallas_call(
        _towers_body,
        grid=grid,
        in_specs=[row_spec, row_spec,
                  full((_DIN, _H1)), full((1, _H1)), full((_H1, _H2)), full((1, _H2)),
                  full((_DIN, _H1)), full((1, _H1)), full((_H1, _H2)), full((1, _H2))],
        out_specs=[out_spec, out_spec],
        out_shape=[jax.ShapeDtypeStruct((_B, _H2), jnp.float32),
                   jax.ShapeDtypeStruct((_B, _H2), jnp.float32)],
    )(ux, ix, wu1, bu1, wu2, bu2, wi1, bi1, wi2, bi2)


def _loss_body(uo, io, lbl, smp, loss):
    u = uo[...]
    it = io[...]
    lblc = lbl[...]
    cols = lax.broadcasted_iota(jnp.int32, (_TB, _B), 1)
    onehot = (cols == lblc).astype(jnp.float32)
    true_w = jnp.dot(onehot, it, preferred_element_type=jnp.float32)
    t = jnp.sum(u * true_w, axis=1, keepdims=True)

    svals = smp[...][:, 0:1]
    scols = lax.broadcasted_iota(jnp.int32, (8, _B), 1)
    smat = (scols == svals).astype(jnp.float32)
    sw = jnp.dot(smat, it, preferred_element_type=jnp.float32)

    m = t
    sls = []
    for j in range(_S):
        slj = jnp.sum(u * sw[j:j + 1, :], axis=1, keepdims=True)
        sls.append(slj)
        m = jnp.maximum(m, slj)
    denom = jnp.exp(t - m)
    for slj in sls:
        denom = denom + jnp.exp(slj - m)
    loss[...] = jnp.log(denom) + m - t


def _loss(uo, io, lbl2, smp):
    grid = (_B // _TB,)
    return pl.pallas_call(
        _loss_body,
        grid=grid,
        in_specs=[pl.BlockSpec((_TB, _H2), lambda t: (t, 0)),
                  pl.BlockSpec((_B, _H2), lambda t: (0, 0)),
                  pl.BlockSpec((_TB, 1), lambda t: (t, 0)),
                  pl.BlockSpec((8, 128), lambda t: (0, 0))],
        out_specs=pl.BlockSpec((_TB, 1), lambda t: (t, 0)),
        out_shape=jax.ShapeDtypeStruct((_B, 1), jnp.float32),
    )(uo, io, lbl2, smp)


def kernel(user_sparse_inputs, user_dense_inputs, item_sparse_inputs,
           item_dense_inputs, labels, user_tables, item_tables,
           W_u1, b_u1, W_u2, b_u2, W_i1, b_i1, W_i2, b_i2):
    info = plsc.get_sparse_core_info()
    nw = info.num_cores * info.num_subcores
    nch = _NCHT // nw
    offs = (jnp.arange(_KSC, dtype=jnp.int32) * _VOCAB)[None, :]
    usp = user_sparse_inputs.astype(jnp.int32)
    isp = item_sparse_inputs.astype(jnp.int32)
    uidx = (usp[:, :_KSC] + offs).reshape(nw, nch, _CHUNK)
    iidx = (isp[:, :_KSC] + offs).reshape(nw, nch, _CHUNK)
    utab = user_tables[:_KSC].reshape(_KSC * _VOCAB, _E)
    itab = item_tables[:_KSC].reshape(_KSC * _VOCAB, _E)

    uemb, iemb = _sc_gather(utab, itab, uidx, iidx)

    def _tc_take(tabs, idx):
        embs = jax.vmap(lambda t, i: jnp.take(t, i, axis=0),
                        in_axes=(0, 1))(tabs, idx)
        return jnp.transpose(embs, (1, 0, 2)).reshape(_B, (_F - _KSC) * _E)

    ux = jnp.concatenate(
        [uemb.reshape(_B, _KSC * _E), _tc_take(user_tables[_KSC:], usp[:, _KSC:])],
        axis=1)
    ix = jnp.concatenate(
        [iemb.reshape(_B, _KSC * _E), _tc_take(item_tables[_KSC:], isp[:, _KSC:])],
        axis=1)

    uo, io = _towers(ux, ix, W_u1, b_u1.reshape(1, _H1), W_u2, b_u2.reshape(1, _H2),
                     W_i1, b_i1.reshape(1, _H1), W_i2, b_i2.reshape(1, _H2))

    sampled = jax.random.randint(jax.random.key(42), (_S,), 0, _B)
    smp = jnp.zeros((8, 128), jnp.int32).at[:_S, 0].set(sampled)
    lbl2 = labels.reshape(_B, 1).astype(jnp.int32)

    loss = _loss(uo, io, lbl2, smp)
    return loss.reshape(_B)


# speedup vs baseline: 1.2104x; 1.0981x over previous
import functools

import jax
import jax.numpy as jnp
from jax import lax
from jax.experimental import pallas as pl
from jax.experimental.pallas import tpu as pltpu
from jax.experimental.pallas import tpu_sc as plsc

_B = 4096
_F = 26
_VOCAB = 100000
_E = 16
_H1, _H2 = 64, 32
_S = 5
_DIN = _F * _E

_CHUNK = 128
_KSC = 14
_ROWS = _B * _KSC
_NCHT = _ROWS // _CHUNK
_NSEM = 8


def _sc_gather_body(nch, utab, itab, uidx, iidx, uout, iout,
                    uidx_v, iidx_v, urows_v, irows_v, sems):
    info = plsc.get_sparse_core_info()
    nc = info.num_cores
    wid = lax.axis_index("s") * nc + lax.axis_index("c")

    pltpu.sync_copy(uidx.at[wid], uidx_v)
    pltpu.sync_copy(iidx.at[wid], iidx_v)

    def fire(j, c):
        pltpu.async_copy(utab.at[uidx_v.at[j]], urows_v.at[j],
                         sems.at[lax.rem(2 * j, _NSEM)])
        pltpu.async_copy(itab.at[iidx_v.at[j]], irows_v.at[j],
                         sems.at[lax.rem(2 * j + 1, _NSEM)])
        return c

    lax.fori_loop(0, nch, fire, 0)

    def drain(j, c):
        pltpu.make_async_copy(utab.at[uidx_v.at[j]], urows_v.at[j],
                              sems.at[lax.rem(2 * j, _NSEM)]).wait()
        pltpu.make_async_copy(itab.at[iidx_v.at[j]], irows_v.at[j],
                              sems.at[lax.rem(2 * j + 1, _NSEM)]).wait()
        return c

    lax.fori_loop(0, nch, drain, 0)

    pltpu.sync_copy(urows_v, uout.at[wid])
    pltpu.sync_copy(irows_v, iout.at[wid])


def _sc_gather(utab, itab, uidx2, iidx2):
    info = plsc.get_sparse_core_info()
    nw = info.num_cores * info.num_subcores
    nch = _NCHT // nw
    mesh = plsc.VectorSubcoreMesh(core_axis_name="c", subcore_axis_name="s")
    out_t = jax.ShapeDtypeStruct((nw, nch, _CHUNK, _E), jnp.float32)
    f = pl.kernel(
        functools.partial(_sc_gather_body, nch),
        out_type=(out_t, out_t),
        mesh=mesh,
        scratch_types=[
            pltpu.VMEM((nch, _CHUNK), jnp.int32),
            pltpu.VMEM((nch, _CHUNK), jnp.int32),
            pltpu.VMEM((nch, _CHUNK, _E), jnp.float32),
            pltpu.VMEM((nch, _CHUNK, _E), jnp.float32),
            pltpu.SemaphoreType.DMA((_NSEM,)),
        ],
        compiler_params=pltpu.CompilerParams(use_tc_tiling_on_sc=False),
    )
    return f(utab, itab, uidx2, iidx2)


_TB = 256


def _towers_body(ux, ix, wu1, bu1, wu2, bu2, wi1, bi1, wi2, bi2, uo, io):
    u = jnp.maximum(jnp.dot(ux[...].astype(jnp.float32), wu1[...],
                            preferred_element_type=jnp.float32) + bu1[...], 0.0)
    u = jnp.maximum(jnp.dot(u, wu2[...],
                            preferred_element_type=jnp.float32) + bu2[...], 0.0)
    uo[...] = u
    it = jnp.maximum(jnp.dot(ix[...].astype(jnp.float32), wi1[...],
                             preferred_element_type=jnp.float32) + bi1[...], 0.0)
    it = jnp.maximum(jnp.dot(it, wi2[...],
                             preferred_element_type=jnp.float32) + bi2[...], 0.0)
    io[...] = it


def _towers(ux, ix, wu1, bu1, wu2, bu2, wi1, bi1, wi2, bi2):
    grid = (_B // _TB,)
    row_spec = pl.BlockSpec((_TB, _DIN), lambda t: (t, 0))
    out_spec = pl.BlockSpec((_TB, _H2), lambda t: (t, 0))

    def full(shape):
        return pl.BlockSpec(shape, lambda t: tuple(0 for _ in shape))

    return pl.pallas_call(
        _towers_body,
        grid=grid,
        in_specs=[row_spec, row_spec,
                  full((_DIN, _H1)), full((1, _H1)), full((_H1, _H2)), full((1, _H2)),
                  full((_DIN, _H1)), full((1, _H1)), full((_H1, _H2)), full((1, _H2))],
        out_specs=[out_spec, out_spec],
        out_shape=[jax.ShapeDtypeStruct((_B, _H2), jnp.float32),
                   jax.ShapeDtypeStruct((_B, _H2), jnp.float32)],
    )(ux, ix, wu1, bu1, wu2, bu2, wi1, bi1, wi2, bi2)


def _loss_body(uo, io, lbl, smp, loss):
    u = uo[...]
    it = io[...]
    lblc = lbl[...]
    cols = lax.broadcasted_iota(jnp.int32, (_TB, _B), 1)
    onehot = (cols == lblc).astype(jnp.float32)
    true_w = jnp.dot(onehot, it, preferred_element_type=jnp.float32)
    t = jnp.sum(u * true_w, axis=1, keepdims=True)

    svals = smp[...][:, 0:1]
    scols = lax.broadcasted_iota(jnp.int32, (8, _B), 1)
    smat = (scols == svals).astype(jnp.float32)
    sw = jnp.dot(smat, it, preferred_element_type=jnp.float32)

    m = t
    sls = []
    for j in range(_S):
        slj = jnp.sum(u * sw[j:j + 1, :], axis=1, keepdims=True)
        sls.append(slj)
        m = jnp.maximum(m, slj)
    denom = jnp.exp(t - m)
    for slj in sls:
        denom = denom + jnp.exp(slj - m)
    loss[...] = jnp.log(denom) + m - t


def _loss(uo, io, lbl2, smp):
    grid = (_B // _TB,)
    return pl.pallas_call(
        _loss_body,
        grid=grid,
        in_specs=[pl.BlockSpec((_TB, _H2), lambda t: (t, 0)),
                  pl.BlockSpec((_B, _H2), lambda t: (0, 0)),
                  pl.BlockSpec((_TB, 1), lambda t: (t, 0)),
                  pl.BlockSpec((8, 128), lambda t: (0, 0))],
        out_specs=pl.BlockSpec((_TB, 1), lambda t: (t, 0)),
        out_shape=jax.ShapeDtypeStruct((_B, 1), jnp.float32),
    )(uo, io, lbl2, smp)


def kernel(user_sparse_inputs, user_dense_inputs, item_sparse_inputs,
           item_dense_inputs, labels, user_tables, item_tables,
           W_u1, b_u1, W_u2, b_u2, W_i1, b_i1, W_i2, b_i2):
    info = plsc.get_sparse_core_info()
    nw = info.num_cores * info.num_subcores
    nch = _NCHT // nw
    offs = (jnp.arange(_KSC, dtype=jnp.int32) * _VOCAB)[None, :]
    usp = user_sparse_inputs.astype(jnp.int32)
    isp = item_sparse_inputs.astype(jnp.int32)
    uidx = (usp[:, :_KSC] + offs).reshape(nw, nch, _CHUNK)
    iidx = (isp[:, :_KSC] + offs).reshape(nw, nch, _CHUNK)
    utab = user_tables[:_KSC].reshape(_KSC * _VOCAB, _E)
    itab = item_tables[:_KSC].reshape(_KSC * _VOCAB, _E)

    uemb, iemb = _sc_gather(utab, itab, uidx, iidx)

    def _tc_take(tabs, idx):
        embs = jax.vmap(lambda t, i: jnp.take(t, i, axis=0),
                        in_axes=(0, 1))(tabs, idx)
        return jnp.transpose(embs, (1, 0, 2)).reshape(_B, (_F - _KSC) * _E)

    ux = jnp.concatenate(
        [uemb.reshape(_B, _KSC * _E), _tc_take(user_tables[_KSC:], usp[:, _KSC:])],
        axis=1)
    ix = jnp.concatenate(
        [iemb.reshape(_B, _KSC * _E), _tc_take(item_tables[_KSC:], isp[:, _KSC:])],
        axis=1)

    uo, io = _towers(ux, ix, W_u1, b_u1.reshape(1, _H1), W_u2, b_u2.reshape(1, _H2),
                     W_i1, b_i1.reshape(1, _H1), W_i2, b_i2.reshape(1, _H2))

    sampled = jax.random.randint(jax.random.key(42), (_S,), 0, _B)
    smp = jnp.zeros((8, 128), jnp.int32).at[:_S, 0].set(sampled)
    lbl2 = labels.reshape(_B, 1).astype(jnp.int32)

    loss = _loss(uo, io, lbl2, smp)
    return loss.reshape(_B)


# speedup vs baseline: 1.3459x; 1.1119x over previous
import functools

import jax
import jax.numpy as jnp
from jax import lax
from jax.experimental import pallas as pl
from jax.experimental.pallas import tpu as pltpu
from jax.experimental.pallas import tpu_sc as plsc

_B = 4096
_F = 26
_VOCAB = 100000
_E = 16
_H1, _H2 = 64, 32
_S = 5
_DIN = _F * _E

_CHUNK = 128
_KSC = 12
_ROWS = _B * _KSC
_NCHT = _ROWS // _CHUNK
_NSEM = 8


def _sc_gather_body(nch, utab, itab, uidx, iidx, uout, iout,
                    uidx_v, iidx_v, urows_v, irows_v, sems):
    info = plsc.get_sparse_core_info()
    nc = info.num_cores
    wid = lax.axis_index("s") * nc + lax.axis_index("c")

    pltpu.sync_copy(uidx.at[wid], uidx_v)
    pltpu.sync_copy(iidx.at[wid], iidx_v)

    def fire(j, c):
        pltpu.async_copy(utab.at[uidx_v.at[j]], urows_v.at[j],
                         sems.at[lax.rem(2 * j, _NSEM)])
        pltpu.async_copy(itab.at[iidx_v.at[j]], irows_v.at[j],
                         sems.at[lax.rem(2 * j + 1, _NSEM)])
        return c

    lax.fori_loop(0, nch, fire, 0)

    def drain(j, c):
        pltpu.make_async_copy(utab.at[uidx_v.at[j]], urows_v.at[j],
                              sems.at[lax.rem(2 * j, _NSEM)]).wait()
        pltpu.make_async_copy(itab.at[iidx_v.at[j]], irows_v.at[j],
                              sems.at[lax.rem(2 * j + 1, _NSEM)]).wait()
        return c

    lax.fori_loop(0, nch, drain, 0)

    pltpu.sync_copy(urows_v, uout.at[wid])
    pltpu.sync_copy(irows_v, iout.at[wid])


def _sc_gather(utab, itab, uidx2, iidx2):
    info = plsc.get_sparse_core_info()
    nw = info.num_cores * info.num_subcores
    nch = _NCHT // nw
    mesh = plsc.VectorSubcoreMesh(core_axis_name="c", subcore_axis_name="s")
    out_t = jax.ShapeDtypeStruct((nw, nch, _CHUNK, _E), jnp.float32)
    f = pl.kernel(
        functools.partial(_sc_gather_body, nch),
        out_type=(out_t, out_t),
        mesh=mesh,
        scratch_types=[
            pltpu.VMEM((nch, _CHUNK), jnp.int32),
            pltpu.VMEM((nch, _CHUNK), jnp.int32),
            pltpu.VMEM((nch, _CHUNK, _E), jnp.float32),
            pltpu.VMEM((nch, _CHUNK, _E), jnp.float32),
            pltpu.SemaphoreType.DMA((_NSEM,)),
        ],
        compiler_params=pltpu.CompilerParams(use_tc_tiling_on_sc=False),
    )
    return f(utab, itab, uidx2, iidx2)


_TB = 256


def _towers_body(ux, ix, wu1, bu1, wu2, bu2, wi1, bi1, wi2, bi2, uo, io):
    u = jnp.maximum(jnp.dot(ux[...].astype(jnp.float32), wu1[...],
                            preferred_element_type=jnp.float32) + bu1[...], 0.0)
    u = jnp.maximum(jnp.dot(u, wu2[...],
                            preferred_element_type=jnp.float32) + bu2[...], 0.0)
    uo[...] = u
    it = jnp.maximum(jnp.dot(ix[...].astype(jnp.float32), wi1[...],
                             preferred_element_type=jnp.float32) + bi1[...], 0.0)
    it = jnp.maximum(jnp.dot(it, wi2[...],
                             preferred_element_type=jnp.float32) + bi2[...], 0.0)
    io[...] = it


def _towers(ux, ix, wu1, bu1, wu2, bu2, wi1, bi1, wi2, bi2):
    grid = (_B // _TB,)
    row_spec = pl.BlockSpec((_TB, _DIN), lambda t: (t, 0))
    out_spec = pl.BlockSpec((_TB, _H2), lambda t: (t, 0))

    def full(shape):
        return pl.BlockSpec(shape, lambda t: tuple(0 for _ in shape))

    return pl.pallas_call(
        _towers_body,
        grid=grid,
        in_specs=[row_spec, row_spec,
                  full((_DIN, _H1)), full((1, _H1)), full((_H1, _H2)), full((1, _H2)),
                  full((_DIN, _H1)), full((1, _H1)), full((_H1, _H2)), full((1, _H2))],
        out_specs=[out_spec, out_spec],
        out_shape=[jax.ShapeDtypeStruct((_B, _H2), jnp.float32),
                   jax.ShapeDtypeStruct((_B, _H2), jnp.float32)],
    )(ux, ix, wu1, bu1, wu2, bu2, wi1, bi1, wi2, bi2)


def _loss_body(uo, io, lbl, smp, loss):
    u = uo[...]
    it = io[...]
    lblc = lbl[...]
    cols = lax.broadcasted_iota(jnp.int32, (_TB, _B), 1)
    onehot = (cols == lblc).astype(jnp.float32)
    true_w = jnp.dot(onehot, it, preferred_element_type=jnp.float32)
    t = jnp.sum(u * true_w, axis=1, keepdims=True)

    svals = smp[...][:, 0:1]
    scols = lax.broadcasted_iota(jnp.int32, (8, _B), 1)
    smat = (scols == svals).astype(jnp.float32)
    sw = jnp.dot(smat, it, preferred_element_type=jnp.float32)

    m = t
    sls = []
    for j in range(_S):
        slj = jnp.sum(u * sw[j:j + 1, :], axis=1, keepdims=True)
        sls.append(slj)
        m = jnp.maximum(m, slj)
    denom = jnp.exp(t - m)
    for slj in sls:
        denom = denom + jnp.exp(slj - m)
    loss[...] = jnp.log(denom) + m - t


def _loss(uo, io, lbl2, smp):
    grid = (_B // _TB,)
    return pl.pallas_call(
        _loss_body,
        grid=grid,
        in_specs=[pl.BlockSpec((_TB, _H2), lambda t: (t, 0)),
                  pl.BlockSpec((_B, _H2), lambda t: (0, 0)),
                  pl.BlockSpec((_TB, 1), lambda t: (t, 0)),
                  pl.BlockSpec((8, 128), lambda t: (0, 0))],
        out_specs=pl.BlockSpec((_TB, 1), lambda t: (t, 0)),
        out_shape=jax.ShapeDtypeStruct((_B, 1), jnp.float32),
    )(uo, io, lbl2, smp)


def kernel(user_sparse_inputs, user_dense_inputs, item_sparse_inputs,
           item_dense_inputs, labels, user_tables, item_tables,
           W_u1, b_u1, W_u2, b_u2, W_i1, b_i1, W_i2, b_i2):
    info = plsc.get_sparse_core_info()
    nw = info.num_cores * info.num_subcores
    nch = _NCHT // nw
    offs = (jnp.arange(_KSC, dtype=jnp.int32) * _VOCAB)[None, :]
    usp = user_sparse_inputs.astype(jnp.int32)
    isp = item_sparse_inputs.astype(jnp.int32)
    uidx = (usp[:, :_KSC] + offs).reshape(nw, nch, _CHUNK)
    iidx = (isp[:, :_KSC] + offs).reshape(nw, nch, _CHUNK)
    utab = user_tables[:_KSC].reshape(_KSC * _VOCAB, _E)
    itab = item_tables[:_KSC].reshape(_KSC * _VOCAB, _E)

    uemb, iemb = _sc_gather(utab, itab, uidx, iidx)

    def _tc_take(tabs, idx):
        embs = jax.vmap(lambda t, i: jnp.take(t, i, axis=0),
                        in_axes=(0, 1))(tabs, idx)
        return jnp.transpose(embs, (1, 0, 2)).reshape(_B, (_F - _KSC) * _E)

    ux = jnp.concatenate(
        [uemb.reshape(_B, _KSC * _E), _tc_take(user_tables[_KSC:], usp[:, _KSC:])],
        axis=1)
    ix = jnp.concatenate(
        [iemb.reshape(_B, _KSC * _E), _tc_take(item_tables[_KSC:], isp[:, _KSC:])],
        axis=1)

    uo, io = _towers(ux, ix, W_u1, b_u1.reshape(1, _H1), W_u2, b_u2.reshape(1, _H2),
                     W_i1, b_i1.reshape(1, _H1), W_i2, b_i2.reshape(1, _H2))

    sampled = jax.random.randint(jax.random.key(42), (_S,), 0, _B)
    smp = jnp.zeros((8, 128), jnp.int32).at[:_S, 0].set(sampled)
    lbl2 = labels.reshape(_B, 1).astype(jnp.int32)

    loss = _loss(uo, io, lbl2, smp)
    return loss.reshape(_B)


# speedup vs baseline: 1.7279x; 1.2838x over previous
import functools

import jax
import jax.numpy as jnp
from jax import lax
from jax.experimental import pallas as pl
from jax.experimental.pallas import tpu as pltpu
from jax.experimental.pallas import tpu_sc as plsc

_B = 4096
_F = 26
_VOCAB = 100000
_E = 16
_H1, _H2 = 64, 32
_S = 5
_DIN = _F * _E

_CHUNK = 128
_KSC = 8
_ROWS = _B * _KSC
_NCHT = _ROWS // _CHUNK
_NSEM = 8


def _sc_gather_body(nch, utab, itab, uidx, iidx, uout, iout,
                    uidx_v, iidx_v, urows_v, irows_v, sems):
    info = plsc.get_sparse_core_info()
    nc = info.num_cores
    wid = lax.axis_index("s") * nc + lax.axis_index("c")

    pltpu.sync_copy(uidx.at[wid], uidx_v)
    pltpu.sync_copy(iidx.at[wid], iidx_v)

    def fire(j, c):
        pltpu.async_copy(utab.at[uidx_v.at[j]], urows_v.at[j],
                         sems.at[lax.rem(2 * j, _NSEM)])
        pltpu.async_copy(itab.at[iidx_v.at[j]], irows_v.at[j],
                         sems.at[lax.rem(2 * j + 1, _NSEM)])
        return c

    lax.fori_loop(0, nch, fire, 0)

    def drain(j, c):
        pltpu.make_async_copy(utab.at[uidx_v.at[j]], urows_v.at[j],
                              sems.at[lax.rem(2 * j, _NSEM)]).wait()
        pltpu.make_async_copy(itab.at[iidx_v.at[j]], irows_v.at[j],
                              sems.at[lax.rem(2 * j + 1, _NSEM)]).wait()
        return c

    lax.fori_loop(0, nch, drain, 0)

    pltpu.sync_copy(urows_v, uout.at[wid])
    pltpu.sync_copy(irows_v, iout.at[wid])


def _sc_gather(utab, itab, uidx2, iidx2):
    info = plsc.get_sparse_core_info()
    nw = info.num_cores * info.num_subcores
    nch = _NCHT // nw
    mesh = plsc.VectorSubcoreMesh(core_axis_name="c", subcore_axis_name="s")
    out_t = jax.ShapeDtypeStruct((nw, nch, _CHUNK, _E), jnp.float32)
    f = pl.kernel(
        functools.partial(_sc_gather_body, nch),
        out_type=(out_t, out_t),
        mesh=mesh,
        scratch_types=[
            pltpu.VMEM((nch, _CHUNK), jnp.int32),
            pltpu.VMEM((nch, _CHUNK), jnp.int32),
            pltpu.VMEM((nch, _CHUNK, _E), jnp.float32),
            pltpu.VMEM((nch, _CHUNK, _E), jnp.float32),
            pltpu.SemaphoreType.DMA((_NSEM,)),
        ],
        compiler_params=pltpu.CompilerParams(use_tc_tiling_on_sc=False),
    )
    return f(utab, itab, uidx2, iidx2)


_TB = 256


def _towers_body(ux, ix, wu1, bu1, wu2, bu2, wi1, bi1, wi2, bi2, uo, io):
    u = jnp.maximum(jnp.dot(ux[...].astype(jnp.float32), wu1[...],
                            preferred_element_type=jnp.float32) + bu1[...], 0.0)
    u = jnp.maximum(jnp.dot(u, wu2[...],
                            preferred_element_type=jnp.float32) + bu2[...], 0.0)
    uo[...] = u
    it = jnp.maximum(jnp.dot(ix[...].astype(jnp.float32), wi1[...],
                             preferred_element_type=jnp.float32) + bi1[...], 0.0)
    it = jnp.maximum(jnp.dot(it, wi2[...],
                             preferred_element_type=jnp.float32) + bi2[...], 0.0)
    io[...] = it


def _towers(ux, ix, wu1, bu1, wu2, bu2, wi1, bi1, wi2, bi2):
    grid = (_B // _TB,)
    row_spec = pl.BlockSpec((_TB, _DIN), lambda t: (t, 0))
    out_spec = pl.BlockSpec((_TB, _H2), lambda t: (t, 0))

    def full(shape):
        return pl.BlockSpec(shape, lambda t: tuple(0 for _ in shape))

    return pl.pallas_call(
        _towers_body,
        grid=grid,
        in_specs=[row_spec, row_spec,
                  full((_DIN, _H1)), full((1, _H1)), full((_H1, _H2)), full((1, _H2)),
                  full((_DIN, _H1)), full((1, _H1)), full((_H1, _H2)), full((1, _H2))],
        out_specs=[out_spec, out_spec],
        out_shape=[jax.ShapeDtypeStruct((_B, _H2), jnp.float32),
                   jax.ShapeDtypeStruct((_B, _H2), jnp.float32)],
    )(ux, ix, wu1, bu1, wu2, bu2, wi1, bi1, wi2, bi2)


def _loss_body(uo, io, lbl, smp, loss):
    u = uo[...]
    it = io[...]
    lblc = lbl[...]
    cols = lax.broadcasted_iota(jnp.int32, (_TB, _B), 1)
    onehot = (cols == lblc).astype(jnp.float32)
    true_w = jnp.dot(onehot, it, preferred_element_type=jnp.float32)
    t = jnp.sum(u * true_w, axis=1, keepdims=True)

    svals = smp[...][:, 0:1]
    scols = lax.broadcasted_iota(jnp.int32, (8, _B), 1)
    smat = (scols == svals).astype(jnp.float32)
    sw = jnp.dot(smat, it, preferred_element_type=jnp.float32)

    m = t
    sls = []
    for j in range(_S):
        slj = jnp.sum(u * sw[j:j + 1, :], axis=1, keepdims=True)
        sls.append(slj)
        m = jnp.maximum(m, slj)
    denom = jnp.exp(t - m)
    for slj in sls:
        denom = denom + jnp.exp(slj - m)
    loss[...] = jnp.log(denom) + m - t


def _loss(uo, io, lbl2, smp):
    grid = (_B // _TB,)
    return pl.pallas_call(
        _loss_body,
        grid=grid,
        in_specs=[pl.BlockSpec((_TB, _H2), lambda t: (t, 0)),
                  pl.BlockSpec((_B, _H2), lambda t: (0, 0)),
                  pl.BlockSpec((_TB, 1), lambda t: (t, 0)),
                  pl.BlockSpec((8, 128), lambda t: (0, 0))],
        out_specs=pl.BlockSpec((_TB, 1), lambda t: (t, 0)),
        out_shape=jax.ShapeDtypeStruct((_B, 1), jnp.float32),
    )(uo, io, lbl2, smp)


def kernel(user_sparse_inputs, user_dense_inputs, item_sparse_inputs,
           item_dense_inputs, labels, user_tables, item_tables,
           W_u1, b_u1, W_u2, b_u2, W_i1, b_i1, W_i2, b_i2):
    info = plsc.get_sparse_core_info()
    nw = info.num_cores * info.num_subcores
    nch = _NCHT // nw
    offs = (jnp.arange(_KSC, dtype=jnp.int32) * _VOCAB)[None, :]
    usp = user_sparse_inputs.astype(jnp.int32)
    isp = item_sparse_inputs.astype(jnp.int32)
    uidx = (usp[:, :_KSC] + offs).reshape(nw, nch, _CHUNK)
    iidx = (isp[:, :_KSC] + offs).reshape(nw, nch, _CHUNK)
    utab = user_tables[:_KSC].reshape(_KSC * _VOCAB, _E)
    itab = item_tables[:_KSC].reshape(_KSC * _VOCAB, _E)

    uemb, iemb = _sc_gather(utab, itab, uidx, iidx)

    def _tc_take(tabs, idx):
        embs = jax.vmap(lambda t, i: jnp.take(t, i, axis=0),
                        in_axes=(0, 1))(tabs, idx)
        return jnp.transpose(embs, (1, 0, 2)).reshape(_B, (_F - _KSC) * _E)

    ux = jnp.concatenate(
        [uemb.reshape(_B, _KSC * _E), _tc_take(user_tables[_KSC:], usp[:, _KSC:])],
        axis=1)
    ix = jnp.concatenate(
        [iemb.reshape(_B, _KSC * _E), _tc_take(item_tables[_KSC:], isp[:, _KSC:])],
        axis=1)

    uo, io = _towers(ux, ix, W_u1, b_u1.reshape(1, _H1), W_u2, b_u2.reshape(1, _H2),
                     W_i1, b_i1.reshape(1, _H1), W_i2, b_i2.reshape(1, _H2))

    sampled = jax.random.randint(jax.random.key(42), (_S,), 0, _B)
    smp = jnp.zeros((8, 128), jnp.int32).at[:_S, 0].set(sampled)
    lbl2 = labels.reshape(_B, 1).astype(jnp.int32)

    loss = _loss(uo, io, lbl2, smp)
    return loss.reshape(_B)
